# baseline (speedup 1.0000x reference)
import functools
from math import sqrt

import jax
import jax.numpy as jnp
from jax import lax
from jax.experimental import pallas as pl
from jax.experimental.pallas import tpu as pltpu
from jax.experimental.pallas import tpu_sc as plsc

_CLUSTERS = 128
_ITERATIONS = 10
_BITS = 32
_BP = 40
_HIGH = lax.Precision.HIGHEST


def _tc_body(q_ref, k_ref, v_ref, planes_ref, bias_ref, sel_ref, vc_ref, assign_ref):
    L = q_ref.shape[1]
    E = q_ref.shape[2] // 2
    C = _CLUSTERS
    q2 = q_ref[0]
    k2 = k_ref[0]
    v2 = v_ref[0]
    sel = sel_ref[...]

    proj2 = jnp.dot(q2, planes_ref[...],
                    preferred_element_type=jnp.float32) + bias_ref[0:1, :]

    lane = lax.broadcasted_iota(jnp.int32, (C, _BP), 1)
    rowc = lax.broadcasted_iota(jnp.int32, (C, _BP), 0).astype(jnp.float32)
    aux = jnp.where(lane == _BITS, -rowc, 0.0)
    is_bit = lane < _BITS
    temp = jnp.float32(1.0 / sqrt(E))

    outs = []
    for j in range(2):
        bits = (proj2[:, j * _BP:(j + 1) * _BP] > 0.0).astype(jnp.float32)
        bpm = bits * 2.0 - 1.0
        cb = jnp.dot(sel, bits, preferred_element_type=jnp.float32)

        def _key_onehot(cb):
            cpm_aug = jnp.where(is_bit, cb * 256.0 - 128.0, aux)
            key = lax.dot_general(cpm_aug, bpm, (((1,), (1,)), ((), ())),
                                  preferred_element_type=jnp.float32)
            m = jnp.max(key, axis=0, keepdims=True)
            return m, (key == m)

        def _iter(_, cb):
            _, hit = _key_onehot(cb)
            bs = jnp.dot(hit.astype(jnp.float32), bits,
                         preferred_element_type=jnp.float32)
            counts = bs[:, _BITS:_BITS + 1]
            maj = (bs * 2.0 > counts).astype(jnp.float32)
            return jnp.where(counts > 0.0, maj, cb)

        cb = lax.fori_loop(0, _ITERATIONS, _iter, cb)

        m, hit = _key_onehot(cb)
        onehot = hit.astype(jnp.float32)
        bs = jnp.dot(onehot, bits, preferred_element_type=jnp.float32)
        counts = jnp.maximum(bs[:, _BITS:_BITS + 1], 1.0)
        assign = (-m.astype(jnp.int32)) % C

        qj = q2[:, j * E:(j + 1) * E]
        kj = k2[:, j * E:(j + 1) * E]
        vj = v2[:, j * E:(j + 1) * E]
        q_sum = lax.dot_general(onehot, qj, (((1,), (0,)), ((), ())),
                                precision=_HIGH,
                                preferred_element_type=jnp.float32)
        qg = q_sum / counts
        logits = lax.dot_general(qg, kj, (((1,), (1,)), ((), ())),
                                 preferred_element_type=jnp.float32) * temp
        lmax = jnp.max(logits, axis=1, keepdims=True)
        p = jnp.exp(logits - lmax)
        a = p / jnp.sum(p, axis=1, keepdims=True)
        vc = jnp.dot(a, vj, preferred_element_type=jnp.float32)
        outs.append((vc, assign))

    vc_ref[0, 0, 0:C, :] = outs[0][0]
    vc_ref[0, 0, C:2 * C, :] = outs[1][0]
    assign_ref[0, 0] = jnp.concatenate([outs[0][1], outs[1][1]], axis=0)


def _tc_cluster_attend(qf, kf, vf, planes2, bias2, sel):
    N, L, HE = qf.shape
    E = 64
    G = HE // 128
    return pl.pallas_call(
        _tc_body,
        grid=(N, G),
        in_specs=[
            pl.BlockSpec((1, L, 128), lambda n, g: (n, 0, g)),
            pl.BlockSpec((1, L, 128), lambda n, g: (n, 0, g)),
            pl.BlockSpec((1, L, 128), lambda n, g: (n, 0, g)),
            pl.BlockSpec((128, 2 * _BP), lambda n, g: (0, 0)),
            pl.BlockSpec((8, 2 * _BP), lambda n, g: (0, 0)),
            pl.BlockSpec((_CLUSTERS, L), lambda n, g: (0, 0)),
        ],
        out_specs=[
            pl.BlockSpec((1, 1, 2 * _CLUSTERS, E), lambda n, g: (n, g, 0, 0)),
            pl.BlockSpec((1, 1, 2, L), lambda n, g: (n, g, 0, 0)),
        ],
        out_shape=[
            jax.ShapeDtypeStruct((N, G, 2 * _CLUSTERS, E), jnp.float32),
            jax.ShapeDtypeStruct((N, G, 2, L), jnp.int32),
        ],
    )(qf, kf, vf, planes2, bias2, sel)


def _make_sc_gather(B, D):
    info = plsc.get_sparse_core_info()
    NC, NS = info.num_cores, info.num_subcores
    NW = NC * NS
    RPW = B // NW
    CH = 128
    NBUF = 4
    nch = RPW // CH
    mesh = plsc.VectorSubcoreMesh(core_axis_name="c", subcore_axis_name="s")

    @functools.partial(
        pl.kernel,
        mesh=mesh,
        compiler_params=pltpu.CompilerParams(use_tc_tiling_on_sc=False),
        out_type=jax.ShapeDtypeStruct((B, D), jnp.float32),
        scratch_types=[
            pltpu.VMEM((RPW,), jnp.int32),
            pltpu.VMEM((NBUF, CH, D), jnp.float32),
        ] + [pltpu.SemaphoreType.DMA] * (2 * NBUF),
    )
    def _sc_gather(table_hbm, idx_hbm, out_hbm, idx_v, rows_v, *sems):
        gsems, wsems = sems[:NBUF], sems[NBUF:]
        wid = lax.axis_index("s") * NC + lax.axis_index("c")
        base = wid * RPW
        pltpu.sync_copy(idx_hbm.at[pl.ds(base, RPW)], idx_v)

        def _start(c):
            return pltpu.async_copy(
                table_hbm.at[idx_v.at[pl.ds(c * CH, CH)]],
                rows_v.at[c % NBUF], gsems[c % NBUF])

        handles = {}
        wh = {}
        for b in range(min(NBUF, nch)):
            handles[b] = _start(b)
        for c in range(nch):
            handles.pop(c).wait()
            wh[c] = pltpu.async_copy(
                rows_v.at[c % NBUF],
                out_hbm.at[pl.ds(base + c * CH, CH)], wsems[c % NBUF])
            nxt = c + NBUF
            if nxt < nch:
                wh.pop(nxt - NBUF).wait()
                handles[nxt] = _start(nxt)
        for c in sorted(wh):
            wh.pop(c).wait()

    return _sc_gather


def kernel(queries, keys, values):
    N, L, H, E = queries.shape
    D = values.shape[-1]
    NH = N * H
    B = N * L * H

    qf = queries.reshape(N, L, H * E)
    kf = keys.reshape(N, L, H * E)
    vf = values.reshape(N, L, H * D)

    planes = jax.random.normal(jax.random.key(42), (_BITS, E + 1), dtype=jnp.float32)
    pad = _BP - _BITS
    planes_aug = jnp.concatenate(
        [planes[:, :-1].T, jnp.zeros((E, pad), jnp.float32)], axis=1)
    planes2 = jnp.zeros((2 * E, 2 * _BP), jnp.float32)
    planes2 = planes2.at[:E, :_BP].set(planes_aug).at[E:, _BP:].set(planes_aug)
    bias_aug = jnp.concatenate([planes[:, -1], jnp.ones((pad,), jnp.float32)])
    bias2 = jnp.tile(jnp.concatenate([bias_aug, bias_aug])[None, :], (8, 1))
    init_idx = jnp.linspace(0, L - 1, _CLUSTERS).astype(jnp.int32)
    sel = (init_idx[:, None] == jnp.arange(L, dtype=jnp.int32)[None, :]
           ).astype(jnp.float32)

    vc, assign = _tc_cluster_attend(qf, kf, vf, planes2, bias2, sel)

    head_off = (jnp.arange(NH, dtype=jnp.int32) * _CLUSTERS).reshape(N, H, 1)
    idx = jnp.transpose(assign.reshape(N, H, L) + head_off, (0, 2, 1)).reshape(B)

    out = _make_sc_gather(B, D)(vc.reshape(NH * _CLUSTERS, D), idx)
    return out.reshape(N, L, H, D)


# speedup vs baseline: 1.0587x; 1.0587x over previous
import functools
from math import sqrt

import jax
import jax.numpy as jnp
from jax import lax
from jax.experimental import pallas as pl
from jax.experimental.pallas import tpu as pltpu
from jax.experimental.pallas import tpu_sc as plsc

_CLUSTERS = 128
_ITERATIONS = 10
_BITS = 32
_BP = 40
_HIGH = lax.Precision.HIGHEST


def _tc_body(q_ref, k_ref, v_ref, planes_ref, bias_ref, sel_ref, vc_ref, assign_ref):
    L = q_ref.shape[1]
    E = q_ref.shape[2] // 2
    C = _CLUSTERS
    q2 = q_ref[0]
    k2 = k_ref[0]
    v2 = v_ref[0]
    sel = sel_ref[...]

    proj2 = jnp.dot(q2, planes_ref[...],
                    preferred_element_type=jnp.float32) + bias_ref[0:1, :]

    lane = lax.broadcasted_iota(jnp.int32, (C, _BP), 1)
    rowc = lax.broadcasted_iota(jnp.int32, (C, _BP), 0).astype(jnp.float32)
    aux = jnp.where(lane == _BITS, -rowc, 0.0)
    is_bit = lane < _BITS
    temp = jnp.float32(1.0 / sqrt(E))

    outs = []
    for j in range(2):
        bits = (proj2[:, j * _BP:(j + 1) * _BP] > 0.0).astype(jnp.float32)
        bpm = bits * 2.0 - 1.0
        cb = jnp.dot(sel, bits, preferred_element_type=jnp.float32)

        def _key_onehot(cb):
            cpm_aug = jnp.where(is_bit, cb * 256.0 - 128.0, aux)
            key = lax.dot_general(cpm_aug, bpm, (((1,), (1,)), ((), ())),
                                  preferred_element_type=jnp.float32)
            m = jnp.max(key, axis=0, keepdims=True)
            return m, (key == m)

        def _iter(_, cb):
            _, hit = _key_onehot(cb)
            bs = jnp.dot(hit.astype(jnp.float32), bits,
                         preferred_element_type=jnp.float32)
            counts = bs[:, _BITS:_BITS + 1]
            maj = (bs * 2.0 > counts).astype(jnp.float32)
            return jnp.where(counts > 0.0, maj, cb)

        cb = lax.fori_loop(0, _ITERATIONS, _iter, cb)

        m, hit = _key_onehot(cb)
        onehot = hit.astype(jnp.float32)
        bs = jnp.dot(onehot, bits, preferred_element_type=jnp.float32)
        counts = jnp.maximum(bs[:, _BITS:_BITS + 1], 1.0)
        assign = (-m.astype(jnp.int32)) % C

        qj = q2[:, j * E:(j + 1) * E]
        kj = k2[:, j * E:(j + 1) * E]
        vj = v2[:, j * E:(j + 1) * E]
        q_hi = qj.astype(jnp.bfloat16).astype(jnp.float32)
        q_lo = qj - q_hi
        dn = (((1,), (0,)), ((), ()))
        q_sum = (lax.dot_general(onehot, q_hi, dn,
                                 preferred_element_type=jnp.float32)
                 + lax.dot_general(onehot, q_lo, dn,
                                   preferred_element_type=jnp.float32))
        qg = q_sum / counts
        logits = lax.dot_general(qg, kj, (((1,), (1,)), ((), ())),
                                 preferred_element_type=jnp.float32) * temp
        lmax = jnp.max(logits, axis=1, keepdims=True)
        p = jnp.exp(logits - lmax)
        a = p / jnp.sum(p, axis=1, keepdims=True)
        vc = jnp.dot(a, vj, preferred_element_type=jnp.float32)
        outs.append((vc, assign))

    vc_ref[0, 0, 0:C, :] = outs[0][0]
    vc_ref[0, 0, C:2 * C, :] = outs[1][0]
    assign_ref[0, 0] = jnp.concatenate([outs[0][1], outs[1][1]], axis=0)


def _tc_cluster_attend(qf, kf, vf, planes2, bias2, sel):
    N, L, HE = qf.shape
    E = 64
    G = HE // 128
    return pl.pallas_call(
        _tc_body,
        grid=(N, G),
        in_specs=[
            pl.BlockSpec((1, L, 128), lambda n, g: (n, 0, g)),
            pl.BlockSpec((1, L, 128), lambda n, g: (n, 0, g)),
            pl.BlockSpec((1, L, 128), lambda n, g: (n, 0, g)),
            pl.BlockSpec((128, 2 * _BP), lambda n, g: (0, 0)),
            pl.BlockSpec((8, 2 * _BP), lambda n, g: (0, 0)),
            pl.BlockSpec((_CLUSTERS, L), lambda n, g: (0, 0)),
        ],
        out_specs=[
            pl.BlockSpec((1, 1, 2 * _CLUSTERS, E), lambda n, g: (n, g, 0, 0)),
            pl.BlockSpec((1, 1, 2, L), lambda n, g: (n, g, 0, 0)),
        ],
        out_shape=[
            jax.ShapeDtypeStruct((N, G, 2 * _CLUSTERS, E), jnp.float32),
            jax.ShapeDtypeStruct((N, G, 2, L), jnp.int32),
        ],
    )(qf, kf, vf, planes2, bias2, sel)


def _make_sc_gather(B, D):
    info = plsc.get_sparse_core_info()
    NC, NS = info.num_cores, info.num_subcores
    NW = NC * NS
    RPW = B // NW
    CH = 128
    NBUF = 4
    nch = RPW // CH
    mesh = plsc.VectorSubcoreMesh(core_axis_name="c", subcore_axis_name="s")

    @functools.partial(
        pl.kernel,
        mesh=mesh,
        compiler_params=pltpu.CompilerParams(use_tc_tiling_on_sc=False),
        out_type=jax.ShapeDtypeStruct((B, D), jnp.float32),
        scratch_types=[
            pltpu.VMEM((RPW,), jnp.int32),
            pltpu.VMEM((NBUF, CH, D), jnp.float32),
        ] + [pltpu.SemaphoreType.DMA] * (2 * NBUF),
    )
    def _sc_gather(table_hbm, idx_hbm, out_hbm, idx_v, rows_v, *sems):
        gsems, wsems = sems[:NBUF], sems[NBUF:]
        wid = lax.axis_index("s") * NC + lax.axis_index("c")
        base = wid * RPW
        pltpu.sync_copy(idx_hbm.at[pl.ds(base, RPW)], idx_v)

        def _start(c):
            return pltpu.async_copy(
                table_hbm.at[idx_v.at[pl.ds(c * CH, CH)]],
                rows_v.at[c % NBUF], gsems[c % NBUF])

        handles = {}
        wh = {}
        for b in range(min(NBUF, nch)):
            handles[b] = _start(b)
        for c in range(nch):
            handles.pop(c).wait()
            wh[c] = pltpu.async_copy(
                rows_v.at[c % NBUF],
                out_hbm.at[pl.ds(base + c * CH, CH)], wsems[c % NBUF])
            nxt = c + NBUF
            if nxt < nch:
                wh.pop(nxt - NBUF).wait()
                handles[nxt] = _start(nxt)
        for c in sorted(wh):
            wh.pop(c).wait()

    return _sc_gather


def kernel(queries, keys, values):
    N, L, H, E = queries.shape
    D = values.shape[-1]
    NH = N * H
    B = N * L * H

    qf = queries.reshape(N, L, H * E)
    kf = keys.reshape(N, L, H * E)
    vf = values.reshape(N, L, H * D)

    planes = jax.random.normal(jax.random.key(42), (_BITS, E + 1), dtype=jnp.float32)
    pad = _BP - _BITS
    planes_aug = jnp.concatenate(
        [planes[:, :-1].T, jnp.zeros((E, pad), jnp.float32)], axis=1)
    planes2 = jnp.zeros((2 * E, 2 * _BP), jnp.float32)
    planes2 = planes2.at[:E, :_BP].set(planes_aug).at[E:, _BP:].set(planes_aug)
    bias_aug = jnp.concatenate([planes[:, -1], jnp.ones((pad,), jnp.float32)])
    bias2 = jnp.tile(jnp.concatenate([bias_aug, bias_aug])[None, :], (8, 1))
    init_idx = jnp.linspace(0, L - 1, _CLUSTERS).astype(jnp.int32)
    sel = (init_idx[:, None] == jnp.arange(L, dtype=jnp.int32)[None, :]
           ).astype(jnp.float32)

    vc, assign = _tc_cluster_attend(qf, kf, vf, planes2, bias2, sel)

    head_off = (jnp.arange(NH, dtype=jnp.int32) * _CLUSTERS).reshape(N, H, 1)
    idx = jnp.transpose(assign.reshape(N, H, L) + head_off, (0, 2, 1)).reshape(B)

    out = _make_sc_gather(B, D)(vc.reshape(NH * _CLUSTERS, D), idx)
    return out.reshape(N, L, H, D)


# speedup vs baseline: 1.1581x; 1.0939x over previous
import functools
from math import sqrt

import jax
import jax.numpy as jnp
from jax import lax
from jax.experimental import pallas as pl
from jax.experimental.pallas import tpu as pltpu
from jax.experimental.pallas import tpu_sc as plsc

_CLUSTERS = 128
_ITERATIONS = 10
_BITS = 32
_BP = 40
_HIGH = lax.Precision.HIGHEST


def _tc_body(q_ref, k_ref, v_ref, planes_ref, bias_ref, sel_ref, vc_ref, assign_ref):
    L = q_ref.shape[1]
    E = q_ref.shape[2] // 2
    C = _CLUSTERS
    q2 = q_ref[0]
    k2 = k_ref[0]
    v2 = v_ref[0]
    sel = sel_ref[...]

    proj2 = jnp.dot(q2, planes_ref[...],
                    preferred_element_type=jnp.float32) + bias_ref[0:1, :]

    lane = lax.broadcasted_iota(jnp.int32, (C, _BP), 1)
    rowc = lax.broadcasted_iota(jnp.int32, (C, _BP), 0).astype(jnp.float32)
    aux = jnp.where(lane == _BITS, -rowc, 0.0)
    is_bit = lane < _BITS
    temp = jnp.float32(1.0 / sqrt(E))

    bits01 = []
    bpm01 = []
    for j in range(2):
        bits = (proj2[:, j * _BP:(j + 1) * _BP] > 0.0).astype(jnp.float32)
        bits01.append(bits)
        bpm01.append(bits * 2.0 - 1.0)

    def _key_onehot(cb, bpm):
        cpm_aug = jnp.where(is_bit, cb * 256.0 - 128.0, aux)
        key = lax.dot_general(cpm_aug, bpm, (((1,), (1,)), ((), ())),
                              preferred_element_type=jnp.float32)
        m = jnp.max(key, axis=0, keepdims=True)
        return m, (key == m)

    def _step(cb, bits, bpm):
        _, hit = _key_onehot(cb, bpm)
        bs = jnp.dot(hit.astype(jnp.float32), bits,
                     preferred_element_type=jnp.float32)
        counts = bs[:, _BITS:_BITS + 1]
        maj = (bs * 2.0 > counts).astype(jnp.float32)
        return jnp.where(counts > 0.0, maj, cb)

    def _iter(_, carry):
        return tuple(_step(carry[j], bits01[j], bpm01[j]) for j in range(2))

    carry = tuple(jnp.dot(sel, bits01[j], preferred_element_type=jnp.float32)
                  for j in range(2))
    carry = lax.fori_loop(0, _ITERATIONS, _iter, carry)

    outs = []
    for j in range(2):
        bits = bits01[j]
        m, hit = _key_onehot(carry[j], bpm01[j])
        onehot = hit.astype(jnp.float32)
        bs = jnp.dot(onehot, bits, preferred_element_type=jnp.float32)
        counts = jnp.maximum(bs[:, _BITS:_BITS + 1], 1.0)
        assign = (-m.astype(jnp.int32)) % C

        qj = q2[:, j * E:(j + 1) * E]
        kj = k2[:, j * E:(j + 1) * E]
        vj = v2[:, j * E:(j + 1) * E]
        q_hi = qj.astype(jnp.bfloat16).astype(jnp.float32)
        q_lo = qj - q_hi
        dn = (((1,), (0,)), ((), ()))
        q_sum = (lax.dot_general(onehot, q_hi, dn,
                                 preferred_element_type=jnp.float32)
                 + lax.dot_general(onehot, q_lo, dn,
                                   preferred_element_type=jnp.float32))
        qg = q_sum / counts
        logits = lax.dot_general(qg, kj, (((1,), (1,)), ((), ())),
                                 preferred_element_type=jnp.float32) * temp
        lmax = jnp.max(logits, axis=1, keepdims=True)
        p = jnp.exp(logits - lmax)
        a = p / jnp.sum(p, axis=1, keepdims=True)
        vc = jnp.dot(a, vj, preferred_element_type=jnp.float32)
        outs.append((vc, assign))

    vc_ref[0, 0, 0:C, :] = outs[0][0]
    vc_ref[0, 0, C:2 * C, :] = outs[1][0]
    assign_ref[0, 0] = jnp.concatenate([outs[0][1], outs[1][1]], axis=0)


def _tc_cluster_attend(qf, kf, vf, planes2, bias2, sel):
    N, L, HE = qf.shape
    E = 64
    G = HE // 128
    return pl.pallas_call(
        _tc_body,
        grid=(N, G),
        in_specs=[
            pl.BlockSpec((1, L, 128), lambda n, g: (n, 0, g)),
            pl.BlockSpec((1, L, 128), lambda n, g: (n, 0, g)),
            pl.BlockSpec((1, L, 128), lambda n, g: (n, 0, g)),
            pl.BlockSpec((128, 2 * _BP), lambda n, g: (0, 0)),
            pl.BlockSpec((8, 2 * _BP), lambda n, g: (0, 0)),
            pl.BlockSpec((_CLUSTERS, L), lambda n, g: (0, 0)),
        ],
        out_specs=[
            pl.BlockSpec((1, 1, 2 * _CLUSTERS, E), lambda n, g: (n, g, 0, 0)),
            pl.BlockSpec((1, 1, 2, L), lambda n, g: (n, g, 0, 0)),
        ],
        out_shape=[
            jax.ShapeDtypeStruct((N, G, 2 * _CLUSTERS, E), jnp.float32),
            jax.ShapeDtypeStruct((N, G, 2, L), jnp.int32),
        ],
    )(qf, kf, vf, planes2, bias2, sel)


def _make_sc_gather(B, D):
    info = plsc.get_sparse_core_info()
    NC, NS = info.num_cores, info.num_subcores
    NW = NC * NS
    RPW = B // NW
    CH = 128
    NBUF = 4
    nch = RPW // CH
    mesh = plsc.VectorSubcoreMesh(core_axis_name="c", subcore_axis_name="s")

    @functools.partial(
        pl.kernel,
        mesh=mesh,
        compiler_params=pltpu.CompilerParams(use_tc_tiling_on_sc=False),
        out_type=jax.ShapeDtypeStruct((B, D), jnp.float32),
        scratch_types=[
            pltpu.VMEM((RPW,), jnp.int32),
            pltpu.VMEM((NBUF, CH, D), jnp.float32),
        ] + [pltpu.SemaphoreType.DMA] * (2 * NBUF),
    )
    def _sc_gather(table_hbm, idx_hbm, out_hbm, idx_v, rows_v, *sems):
        gsems, wsems = sems[:NBUF], sems[NBUF:]
        wid = lax.axis_index("s") * NC + lax.axis_index("c")
        base = wid * RPW
        pltpu.sync_copy(idx_hbm.at[pl.ds(base, RPW)], idx_v)

        def _start(c):
            return pltpu.async_copy(
                table_hbm.at[idx_v.at[pl.ds(c * CH, CH)]],
                rows_v.at[c % NBUF], gsems[c % NBUF])

        handles = {}
        wh = {}
        for b in range(min(NBUF, nch)):
            handles[b] = _start(b)
        for c in range(nch):
            handles.pop(c).wait()
            wh[c] = pltpu.async_copy(
                rows_v.at[c % NBUF],
                out_hbm.at[pl.ds(base + c * CH, CH)], wsems[c % NBUF])
            nxt = c + NBUF
            if nxt < nch:
                wh.pop(nxt - NBUF).wait()
                handles[nxt] = _start(nxt)
        for c in sorted(wh):
            wh.pop(c).wait()

    return _sc_gather


def kernel(queries, keys, values):
    N, L, H, E = queries.shape
    D = values.shape[-1]
    NH = N * H
    B = N * L * H

    qf = queries.reshape(N, L, H * E)
    kf = keys.reshape(N, L, H * E)
    vf = values.reshape(N, L, H * D)

    planes = jax.random.normal(jax.random.key(42), (_BITS, E + 1), dtype=jnp.float32)
    pad = _BP - _BITS
    planes_aug = jnp.concatenate(
        [planes[:, :-1].T, jnp.zeros((E, pad), jnp.float32)], axis=1)
    planes2 = jnp.zeros((2 * E, 2 * _BP), jnp.float32)
    planes2 = planes2.at[:E, :_BP].set(planes_aug).at[E:, _BP:].set(planes_aug)
    bias_aug = jnp.concatenate([planes[:, -1], jnp.ones((pad,), jnp.float32)])
    bias2 = jnp.tile(jnp.concatenate([bias_aug, bias_aug])[None, :], (8, 1))
    init_idx = jnp.linspace(0, L - 1, _CLUSTERS).astype(jnp.int32)
    sel = (init_idx[:, None] == jnp.arange(L, dtype=jnp.int32)[None, :]
           ).astype(jnp.float32)

    vc, assign = _tc_cluster_attend(qf, kf, vf, planes2, bias2, sel)

    head_off = (jnp.arange(NH, dtype=jnp.int32) * _CLUSTERS).reshape(N, H, 1)
    idx = jnp.transpose(assign.reshape(N, H, L) + head_off, (0, 2, 1)).reshape(B)

    out = _make_sc_gather(B, D)(vc.reshape(NH * _CLUSTERS, D), idx)
    return out.reshape(N, L, H, D)


# speedup vs baseline: 1.1622x; 1.0035x over previous
import functools
from math import sqrt

import jax
import jax.numpy as jnp
from jax import lax
from jax.experimental import pallas as pl
from jax.experimental.pallas import tpu as pltpu
from jax.experimental.pallas import tpu_sc as plsc

_CLUSTERS = 128
_ITERATIONS = 10
_BITS = 32
_BP = 40
_HIGH = lax.Precision.HIGHEST


def _tc_body(q_ref, k_ref, v_ref, planes_ref, bias_ref, sel_ref, vc_ref, assign_ref):
    L = q_ref.shape[1]
    E = q_ref.shape[2] // 2
    C = _CLUSTERS
    q2 = q_ref[0]
    k2 = k_ref[0]
    v2 = v_ref[0]
    sel = sel_ref[...]

    proj2 = jnp.dot(q2, planes_ref[...],
                    preferred_element_type=jnp.float32) + bias_ref[0:1, :]

    lane = lax.broadcasted_iota(jnp.int32, (C, _BP), 1)
    rowc = lax.broadcasted_iota(jnp.int32, (C, _BP), 0).astype(jnp.float32)
    aux = jnp.where(lane == _BITS, -rowc, 0.0)
    is_bit = lane < _BITS
    temp = jnp.float32(1.0 / sqrt(E))

    bits01 = []
    bpm01 = []
    for j in range(2):
        bits = (proj2[:, j * _BP:(j + 1) * _BP] > 0.0).astype(jnp.float32)
        bits01.append(bits)
        bpm01.append(bits * 2.0 - 1.0)

    def _key_onehot(cb, bpm):
        cpm_aug = jnp.where(is_bit, cb * 256.0 - 128.0, aux)
        key = lax.dot_general(cpm_aug, bpm, (((1,), (1,)), ((), ())),
                              preferred_element_type=jnp.float32)
        m = jnp.max(key, axis=0, keepdims=True)
        return m, (key == m)

    def _step(cb, bits, bpm):
        _, hit = _key_onehot(cb, bpm)
        bs = jnp.dot(hit.astype(jnp.float32), bits,
                     preferred_element_type=jnp.float32)
        counts = bs[:, _BITS:_BITS + 1]
        maj = (bs * 2.0 > counts).astype(jnp.float32)
        return jnp.where(counts > 0.0, maj, cb)

    def _iter(_, carry):
        return tuple(_step(carry[j], bits01[j], bpm01[j]) for j in range(2))

    carry = tuple(jnp.dot(sel, bits01[j], preferred_element_type=jnp.float32)
                  for j in range(2))
    carry = lax.fori_loop(0, _ITERATIONS, _iter, carry)

    q2_hi = q2.astype(jnp.bfloat16).astype(jnp.float32)
    q2_lo = q2 - q2_hi
    lane2 = lax.broadcasted_iota(jnp.int32, (C, 2 * E), 1)
    dn = (((1,), (0,)), ((), ()))
    outs = []
    for j in range(2):
        bits = bits01[j]
        m, hit = _key_onehot(carry[j], bpm01[j])
        onehot = hit.astype(jnp.float32)
        bs = jnp.dot(onehot, bits, preferred_element_type=jnp.float32)
        counts = jnp.maximum(bs[:, _BITS:_BITS + 1], 1.0)
        assign = (-m.astype(jnp.int32)) % C

        q_sum2 = (lax.dot_general(onehot, q2_hi, dn,
                                  preferred_element_type=jnp.float32)
                  + lax.dot_general(onehot, q2_lo, dn,
                                    preferred_element_type=jnp.float32))
        in_head = (lane2 // E) == j
        qg_pad = jnp.where(in_head, q_sum2, 0.0) / counts
        logits = lax.dot_general(qg_pad, k2, (((1,), (1,)), ((), ())),
                                 preferred_element_type=jnp.float32) * temp
        lmax = jnp.max(logits, axis=1, keepdims=True)
        p = jnp.exp(logits - lmax)
        a = p / jnp.sum(p, axis=1, keepdims=True)
        vc2 = jnp.dot(a, v2, preferred_element_type=jnp.float32)
        outs.append((vc2[:, j * E:(j + 1) * E], assign))

    vc_ref[0, 0, 0:C, :] = outs[0][0]
    vc_ref[0, 0, C:2 * C, :] = outs[1][0]
    assign_ref[0, 0] = jnp.concatenate([outs[0][1], outs[1][1]], axis=0)


def _tc_cluster_attend(qf, kf, vf, planes2, bias2, sel):
    N, L, HE = qf.shape
    E = 64
    G = HE // 128
    return pl.pallas_call(
        _tc_body,
        grid=(N, G),
        in_specs=[
            pl.BlockSpec((1, L, 128), lambda n, g: (n, 0, g)),
            pl.BlockSpec((1, L, 128), lambda n, g: (n, 0, g)),
            pl.BlockSpec((1, L, 128), lambda n, g: (n, 0, g)),
            pl.BlockSpec((128, 2 * _BP), lambda n, g: (0, 0)),
            pl.BlockSpec((8, 2 * _BP), lambda n, g: (0, 0)),
            pl.BlockSpec((_CLUSTERS, L), lambda n, g: (0, 0)),
        ],
        out_specs=[
            pl.BlockSpec((1, 1, 2 * _CLUSTERS, E), lambda n, g: (n, g, 0, 0)),
            pl.BlockSpec((1, 1, 2, L), lambda n, g: (n, g, 0, 0)),
        ],
        out_shape=[
            jax.ShapeDtypeStruct((N, G, 2 * _CLUSTERS, E), jnp.float32),
            jax.ShapeDtypeStruct((N, G, 2, L), jnp.int32),
        ],
    )(qf, kf, vf, planes2, bias2, sel)


def _make_sc_gather(B, D):
    info = plsc.get_sparse_core_info()
    NC, NS = info.num_cores, info.num_subcores
    NW = NC * NS
    RPW = B // NW
    CH = 128
    NBUF = 4
    nch = RPW // CH
    mesh = plsc.VectorSubcoreMesh(core_axis_name="c", subcore_axis_name="s")

    @functools.partial(
        pl.kernel,
        mesh=mesh,
        compiler_params=pltpu.CompilerParams(use_tc_tiling_on_sc=False),
        out_type=jax.ShapeDtypeStruct((B, D), jnp.float32),
        scratch_types=[
            pltpu.VMEM((RPW,), jnp.int32),
            pltpu.VMEM((NBUF, CH, D), jnp.float32),
        ] + [pltpu.SemaphoreType.DMA] * (2 * NBUF),
    )
    def _sc_gather(table_hbm, idx_hbm, out_hbm, idx_v, rows_v, *sems):
        gsems, wsems = sems[:NBUF], sems[NBUF:]
        wid = lax.axis_index("s") * NC + lax.axis_index("c")
        base = wid * RPW
        pltpu.sync_copy(idx_hbm.at[pl.ds(base, RPW)], idx_v)

        def _start(c):
            return pltpu.async_copy(
                table_hbm.at[idx_v.at[pl.ds(c * CH, CH)]],
                rows_v.at[c % NBUF], gsems[c % NBUF])

        handles = {}
        wh = {}
        for b in range(min(NBUF, nch)):
            handles[b] = _start(b)
        for c in range(nch):
            handles.pop(c).wait()
            wh[c] = pltpu.async_copy(
                rows_v.at[c % NBUF],
                out_hbm.at[pl.ds(base + c * CH, CH)], wsems[c % NBUF])
            nxt = c + NBUF
            if nxt < nch:
                wh.pop(nxt - NBUF).wait()
                handles[nxt] = _start(nxt)
        for c in sorted(wh):
            wh.pop(c).wait()

    return _sc_gather


def kernel(queries, keys, values):
    N, L, H, E = queries.shape
    D = values.shape[-1]
    NH = N * H
    B = N * L * H

    qf = queries.reshape(N, L, H * E)
    kf = keys.reshape(N, L, H * E)
    vf = values.reshape(N, L, H * D)

    planes = jax.random.normal(jax.random.key(42), (_BITS, E + 1), dtype=jnp.float32)
    pad = _BP - _BITS
    planes_aug = jnp.concatenate(
        [planes[:, :-1].T, jnp.zeros((E, pad), jnp.float32)], axis=1)
    planes2 = jnp.zeros((2 * E, 2 * _BP), jnp.float32)
    planes2 = planes2.at[:E, :_BP].set(planes_aug).at[E:, _BP:].set(planes_aug)
    bias_aug = jnp.concatenate([planes[:, -1], jnp.ones((pad,), jnp.float32)])
    bias2 = jnp.tile(jnp.concatenate([bias_aug, bias_aug])[None, :], (8, 1))
    init_idx = jnp.linspace(0, L - 1, _CLUSTERS).astype(jnp.int32)
    sel = (init_idx[:, None] == jnp.arange(L, dtype=jnp.int32)[None, :]
           ).astype(jnp.float32)

    vc, assign = _tc_cluster_attend(qf, kf, vf, planes2, bias2, sel)

    head_off = (jnp.arange(NH, dtype=jnp.int32) * _CLUSTERS).reshape(N, H, 1)
    idx = jnp.transpose(assign.reshape(N, H, L) + head_off, (0, 2, 1)).reshape(B)

    out = _make_sc_gather(B, D)(vc.reshape(NH * _CLUSTERS, D), idx)
    return out.reshape(N, L, H, D)
